# Initial kernel scaffold; baseline (speedup 1.0000x reference)
import jax
import jax.numpy as jnp
from jax.experimental import pallas as pl


def kernel(logits, targets):
    raise NotImplementedError("write your pallas kernel here")


# speedup vs baseline: 2.2252x; 2.2252x over previous
import functools

import jax
import jax.numpy as jnp
from jax import lax
from jax.experimental import pallas as pl
from jax.experimental.pallas import tpu as pltpu
from jax.experimental.pallas import tpu_sc as plsc

_B = 128
_N = 100000
_K = 30
_PRESENCE_W = 2.0
_ABSENCE_W = 1.0
_TOPK_BONUS_W = 0.05

_NC = 2
_NS = 16
_NW = _NC * _NS
_TASKS = 2 * _B
_TPW = _TASKS // _NW
_HALF = _N // 2
_NVH = _HALF // 16
_NV = _N // 16
_CAP = 4096


def _sc_body(logits_hbm, targets_hbm, out_hbm, row_v, cand_v, res_v):
    cid = lax.axis_index("c")
    sid = lax.axis_index("s")
    wid = sid * _NC + cid
    lane = lax.iota(jnp.int32, 16)
    ninf = jnp.full((16,), -jnp.inf, jnp.float32)
    res = jnp.zeros((16,), jnp.float32)

    for j in range(_TPW):
        src = logits_hbm if j < _TPW // 2 else targets_hbm
        row = j * _NW + wid - (_B if j >= _TPW // 2 else 0)
        pltpu.sync_copy(src.at[row], row_v)

        def amax(i, accs):
            a0, a1 = accs
            a0 = jnp.maximum(a0, row_v[pl.ds(i * 16, 16)])
            a1 = jnp.maximum(a1, row_v[pl.ds(_HALF + i * 16, 16)])
            return a0, a1

        a0, a1 = lax.fori_loop(0, _NVH, amax, (ninf, ninf))
        theta_c = jnp.min(jnp.minimum(a0, a1))

        def filt(i, off):
            v = row_v[pl.ds(i * 16, 16)]
            m = v >= theta_c
            o = jnp.minimum(off, _CAP - 16)
            plsc.store_compressed(cand_v.at[pl.ds(o, 16)], v, mask=m)
            return off + jnp.sum(m.astype(jnp.int32))

        c = lax.fori_loop(0, _NV, filt, jnp.int32(0))
        c = jnp.minimum(c, _CAP - 16)
        cand_v[pl.ds(c, 16)] = ninf
        nv = (c + 15) // 16

        def cond(st):
            return st[0] > 0

        def body(st):
            r, ans = st

            def mx_it(i, acc):
                return jnp.maximum(acc, cand_v[pl.ds(i * 16, 16)])

            mx = jnp.max(lax.fori_loop(0, nv, mx_it, ninf))

            def cnt_it(i, tot):
                v = cand_v[pl.ds(i * 16, 16)]
                e = v == mx
                cand_v[pl.ds(i * 16, 16)] = jnp.where(e, ninf, v)
                return tot + jnp.sum(e.astype(jnp.int32))

            cnt = lax.fori_loop(0, nv, cnt_it, jnp.int32(0))
            done = cnt >= r
            ans = jnp.where(done, mx, ans)
            r = jnp.where(done, jnp.int32(0), r - cnt)
            return r, ans

        _, thr = lax.while_loop(cond, body, (jnp.int32(_K), jnp.float32(0.0)))
        res = jnp.where(lane == j, thr, res)

    res_v[...] = res
    pltpu.sync_copy(res_v, out_hbm.at[wid])


@functools.partial(
    pl.kernel,
    mesh=plsc.VectorSubcoreMesh(core_axis_name="c", subcore_axis_name="s"),
    out_type=jax.ShapeDtypeStruct((_NW, 16), jnp.float32),
    compiler_params=pltpu.CompilerParams(needs_layout_passes=False),
    scratch_types=[
        pltpu.VMEM((_N,), jnp.float32),
        pltpu.VMEM((_CAP,), jnp.float32),
        pltpu.VMEM((16,), jnp.float32),
    ],
)
def _sc_thresholds(logits_hbm, targets_hbm, out_hbm, row_v, cand_v, res_v):
    _sc_body(logits_hbm, targets_hbm, out_hbm, row_v, cand_v, res_v)


_BR = 8


def _tc_body(lg_ref, tg_ref, thl_ref, tht_ref, s_ref, c_ref):
    i = pl.program_id(0)
    x = lg_ref[...]
    t = tg_ref[...]
    thl = thl_ref[...]
    tht = tht_ref[...]

    bce = jnp.maximum(x, 0.0) - x * t + jnp.log1p(jnp.exp(-jnp.abs(x)))
    tmask = (t >= tht).astype(jnp.float32)
    w = jnp.where(t > 0.5, _PRESENCE_W, _ABSENCE_W) + 0.5 * tmask
    ov = tmask * (x >= thl).astype(jnp.float32)
    s = jnp.sum(bce * w)
    cv = jnp.sum(ov)

    @pl.when(i == 0)
    def _():
        s_ref[...] = jnp.zeros((1, 1), jnp.float32)
        c_ref[...] = jnp.zeros((1, 1), jnp.float32)

    s_ref[...] += jnp.full((1, 1), 0.0) + s
    c_ref[...] += jnp.full((1, 1), 0.0) + cv


def _tc_combine(logits, targets, thl, tht):
    return pl.pallas_call(
        _tc_body,
        grid=(_B // _BR,),
        in_specs=[
            pl.BlockSpec((_BR, _N), lambda i: (i, 0)),
            pl.BlockSpec((_BR, _N), lambda i: (i, 0)),
            pl.BlockSpec((_BR, 1), lambda i: (i, 0)),
            pl.BlockSpec((_BR, 1), lambda i: (i, 0)),
        ],
        out_specs=[
            pl.BlockSpec((1, 1), lambda i: (0, 0)),
            pl.BlockSpec((1, 1), lambda i: (0, 0)),
        ],
        out_shape=[
            jax.ShapeDtypeStruct((1, 1), jnp.float32),
            jax.ShapeDtypeStruct((1, 1), jnp.float32),
        ],
    )(logits, targets, thl, tht)


def kernel(logits, targets):
    thr = _sc_thresholds(logits, targets)
    flat = thr.T.reshape(-1)[:_TASKS]
    thl = flat[:_B].reshape(_B, 1)
    tht = flat[_B:].reshape(_B, 1)
    s, c = _tc_combine(logits, targets, thl, tht)
    bce_loss = s[0, 0] / (_B * _N)
    topk_loss = 1.0 - c[0, 0] / (_K * _B)
    return bce_loss + _TOPK_BONUS_W * topk_loss


# speedup vs baseline: 4.1953x; 1.8854x over previous
import functools

import jax
import jax.numpy as jnp
from jax import lax
from jax.experimental import pallas as pl
from jax.experimental.pallas import tpu as pltpu
from jax.experimental.pallas import tpu_sc as plsc

_B = 128
_N = 100000
_K = 30
_PRESENCE_W = 2.0
_ABSENCE_W = 1.0
_TOPK_BONUS_W = 0.05

_NC = 2
_NS = 16
_NW = _NC * _NS
_TASKS = 2 * _B
_TPW = _TASKS // _NW
_NO = 10
_NG = 625
_CAP = 4096


def _sc_body(logits_hbm, targets_hbm, out_hbm, row_v, max_v, cand_v, res_v):
    cid = lax.axis_index("c")
    sid = lax.axis_index("s")
    wid = sid * _NC + cid
    lane = lax.iota(jnp.int32, 16)
    ninf = jnp.full((16,), -jnp.inf, jnp.float32)
    res = jnp.zeros((16,), jnp.float32)

    for j in range(_TPW):
        src = logits_hbm if j < _TPW // 2 else targets_hbm
        row = j * _NW + wid - (_B if j >= _TPW // 2 else 0)
        pltpu.sync_copy(src.at[row], row_v)

        def amax2(g, accs):
            a0, a1 = accs
            lo = None
            hi = None
            for o in range(_NO):
                v = row_v[pl.ds(o * (_NG * 16) + g * 16, 16)]
                if o < _NO // 2:
                    lo = v if lo is None else jnp.maximum(lo, v)
                else:
                    hi = v if hi is None else jnp.maximum(hi, v)

            max_v[pl.ds(g * 16, 16)] = jnp.maximum(lo, hi)
            return jnp.maximum(a0, lo), jnp.maximum(a1, hi)

        a0, a1 = lax.fori_loop(0, _NG, amax2, (ninf, ninf))
        theta_c = jnp.min(jnp.minimum(a0, a1))

        def collect(g, off):
            m = max_v[pl.ds(g * 16, 16)] >= theta_c
            nhit = plsc.all_reduce_population_count(m)[0]

            def hit_cond(st):
                return st[0] > 0

            def hit_body(st):
                n, mm, o = st
                l = plsc.all_reduce_ffs(mm)[0]
                o_lane = jnp.minimum(lane, _NO - 1)
                idx = (o_lane * _NG + g) * 16 + l
                vals = plsc.load_gather(row_v, [idx])
                m2 = jnp.logical_and(lane < _NO, vals >= theta_c)
                o_use = jnp.minimum(o, _CAP - 16)
                plsc.store_compressed(cand_v.at[pl.ds(o_use, 16)], vals, mask=m2)
                o = o + plsc.all_reduce_population_count(m2)[0]
                mm = jnp.logical_and(mm, lane != l)
                return n - 1, mm, o

            _, _, off = lax.while_loop(hit_cond, hit_body, (nhit, m, off))
            return off

        c = lax.fori_loop(0, _NG, collect, jnp.int32(0))
        c = jnp.minimum(c, _CAP - 16)
        cand_v[pl.ds(c, 16)] = ninf
        nv = (c + 15) // 16

        def cond(st):
            return st[0] > 0

        def body(st):
            r, ans = st

            def mx_it(i, acc):
                return jnp.maximum(acc, cand_v[pl.ds(i * 16, 16)])

            mx = jnp.max(lax.fori_loop(0, nv, mx_it, ninf))

            def cnt_it(i, tot):
                v = cand_v[pl.ds(i * 16, 16)]
                e = v == mx
                cand_v[pl.ds(i * 16, 16)] = jnp.where(e, ninf, v)
                return tot + jnp.sum(e.astype(jnp.int32))

            cnt = lax.fori_loop(0, nv, cnt_it, jnp.int32(0))
            done = cnt >= r
            ans = jnp.where(done, mx, ans)
            r = jnp.where(done, jnp.int32(0), r - cnt)
            return r, ans

        _, thr = lax.while_loop(cond, body, (jnp.int32(_K), jnp.float32(0.0)))
        res = jnp.where(lane == j, thr, res)

    res_v[...] = res
    pltpu.sync_copy(res_v, out_hbm.at[wid])


@functools.partial(
    pl.kernel,
    mesh=plsc.VectorSubcoreMesh(core_axis_name="c", subcore_axis_name="s"),
    out_type=jax.ShapeDtypeStruct((_NW, 16), jnp.float32),
    compiler_params=pltpu.CompilerParams(needs_layout_passes=False),
    scratch_types=[
        pltpu.VMEM((_N,), jnp.float32),
        pltpu.VMEM((_NG * 16,), jnp.float32),
        pltpu.VMEM((_CAP,), jnp.float32),
        pltpu.VMEM((16,), jnp.float32),
    ],
)
def _sc_thresholds(logits_hbm, targets_hbm, out_hbm, row_v, max_v, cand_v, res_v):
    _sc_body(logits_hbm, targets_hbm, out_hbm, row_v, max_v, cand_v, res_v)


_BR = 8


def _tc_body(lg_ref, tg_ref, thl_ref, tht_ref, s_ref, c_ref):
    i = pl.program_id(0)
    x = lg_ref[...]
    t = tg_ref[...]
    thl = thl_ref[...]
    tht = tht_ref[...]

    bce = jnp.maximum(x, 0.0) - x * t + jnp.log1p(jnp.exp(-jnp.abs(x)))
    tmask = (t >= tht).astype(jnp.float32)
    w = jnp.where(t > 0.5, _PRESENCE_W, _ABSENCE_W) + 0.5 * tmask
    ov = tmask * (x >= thl).astype(jnp.float32)
    s = jnp.sum(bce * w)
    cv = jnp.sum(ov)

    @pl.when(i == 0)
    def _():
        s_ref[...] = jnp.zeros((1, 1), jnp.float32)
        c_ref[...] = jnp.zeros((1, 1), jnp.float32)

    s_ref[...] += jnp.full((1, 1), 0.0) + s
    c_ref[...] += jnp.full((1, 1), 0.0) + cv


def _tc_combine(logits, targets, thl, tht):
    return pl.pallas_call(
        _tc_body,
        grid=(_B // _BR,),
        in_specs=[
            pl.BlockSpec((_BR, _N), lambda i: (i, 0)),
            pl.BlockSpec((_BR, _N), lambda i: (i, 0)),
            pl.BlockSpec((_BR, 1), lambda i: (i, 0)),
            pl.BlockSpec((_BR, 1), lambda i: (i, 0)),
        ],
        out_specs=[
            pl.BlockSpec((1, 1), lambda i: (0, 0)),
            pl.BlockSpec((1, 1), lambda i: (0, 0)),
        ],
        out_shape=[
            jax.ShapeDtypeStruct((1, 1), jnp.float32),
            jax.ShapeDtypeStruct((1, 1), jnp.float32),
        ],
    )(logits, targets, thl, tht)


def kernel(logits, targets):
    thr = _sc_thresholds(logits, targets)
    flat = thr.T.reshape(-1)[:_TASKS]
    thl = flat[:_B].reshape(_B, 1)
    tht = flat[_B:].reshape(_B, 1)
    s, c = _tc_combine(logits, targets, thl, tht)
    bce_loss = s[0, 0] / (_B * _N)
    topk_loss = 1.0 - c[0, 0] / (_K * _B)
    return bce_loss + _TOPK_BONUS_W * topk_loss


# speedup vs baseline: 5.2215x; 1.2446x over previous
import functools

import jax
import jax.numpy as jnp
from jax import lax
from jax.experimental import pallas as pl
from jax.experimental.pallas import tpu as pltpu
from jax.experimental.pallas import tpu_sc as plsc

_B = 128
_N = 100000
_K = 30
_PRESENCE_W = 2.0
_ABSENCE_W = 1.0
_TOPK_BONUS_W = 0.05

_NC = 2
_NS = 16
_NW = _NC * _NS
_TASKS = 2 * _B
_TPW = _TASKS // _NW
_NO = 10
_NG = 625
_CAP = 4096
_WCAP = 1024


def _sc_body(logits_hbm, targets_hbm, out_hbm, row_v, max_v, cand_v, work_v, res_v):
    cid = lax.axis_index("c")
    sid = lax.axis_index("s")
    wid = sid * _NC + cid
    lane = lax.iota(jnp.int32, 16)
    ninf = jnp.full((16,), -jnp.inf, jnp.float32)
    res = jnp.zeros((16,), jnp.float32)

    for j in range(_TPW):
        src = logits_hbm if j < _TPW // 2 else targets_hbm
        row = j * _NW + wid - (_B if j >= _TPW // 2 else 0)
        pltpu.sync_copy(src.at[row], row_v)

        def amax2(g, accs):
            a0, a1 = accs
            lo = None
            hi = None
            for o in range(_NO):
                v = row_v[pl.ds(o * (_NG * 16) + g * 16, 16)]
                if o < _NO // 2:
                    lo = v if lo is None else jnp.maximum(lo, v)
                else:
                    hi = v if hi is None else jnp.maximum(hi, v)

            max_v[pl.ds(g * 16, 16)] = jnp.maximum(lo, hi)
            return jnp.maximum(a0, lo), jnp.maximum(a1, hi)

        a0, a1 = lax.fori_loop(0, _NG, amax2, (ninf, ninf))
        theta_c = jnp.min(jnp.minimum(a0, a1))

        def scan_flags(g, noff):
            m = max_v[pl.ds(g * 16, 16)] >= theta_c
            cid = g * 16 + lane
            o_use = jnp.minimum(noff, _WCAP - 16)
            plsc.store_compressed(work_v.at[pl.ds(o_use, 16)], cid, mask=m)
            return noff + plsc.all_reduce_population_count(m)[0]

        nw = lax.fori_loop(0, _NG, scan_flags, jnp.int32(0))
        nw = jnp.minimum(nw, _WCAP - 16)

        def gather_chunks(i, coff):
            w = work_v[pl.ds(i * 16, 16)]
            valid = (i * 16 + lane) < nw
            w = jnp.minimum(jnp.maximum(w, 0), _NG * 16 - 1)
            for o in range(_NO):
                vals = plsc.load_gather(row_v, [w + o * (_NG * 16)])
                m2 = jnp.logical_and(valid, vals >= theta_c)
                o_use = jnp.minimum(coff, _CAP - 16)
                plsc.store_compressed(cand_v.at[pl.ds(o_use, 16)], vals, mask=m2)
                coff = coff + plsc.all_reduce_population_count(m2)[0]
            return coff

        c = lax.fori_loop(0, (nw + 15) // 16, gather_chunks, jnp.int32(0))
        c = jnp.minimum(c, _CAP - 16)
        cand_v[pl.ds(c, 16)] = ninf
        nv = (c + 15) // 16

        def cond(st):
            return st[0] > 0

        def body(st):
            r, ans = st

            def mx_it(i, acc):
                return jnp.maximum(acc, cand_v[pl.ds(i * 16, 16)])

            mx = jnp.max(lax.fori_loop(0, nv, mx_it, ninf))

            def cnt_it(i, tot):
                v = cand_v[pl.ds(i * 16, 16)]
                e = v == mx
                cand_v[pl.ds(i * 16, 16)] = jnp.where(e, ninf, v)
                return tot + jnp.sum(e.astype(jnp.int32))

            cnt = lax.fori_loop(0, nv, cnt_it, jnp.int32(0))
            done = cnt >= r
            ans = jnp.where(done, mx, ans)
            r = jnp.where(done, jnp.int32(0), r - cnt)
            return r, ans

        _, thr = lax.while_loop(cond, body, (jnp.int32(_K), jnp.float32(0.0)))
        res = jnp.where(lane == j, thr, res)

    res_v[...] = res
    pltpu.sync_copy(res_v, out_hbm.at[wid])


@functools.partial(
    pl.kernel,
    mesh=plsc.VectorSubcoreMesh(core_axis_name="c", subcore_axis_name="s"),
    out_type=jax.ShapeDtypeStruct((_NW, 16), jnp.float32),
    compiler_params=pltpu.CompilerParams(needs_layout_passes=False),
    scratch_types=[
        pltpu.VMEM((_N,), jnp.float32),
        pltpu.VMEM((_NG * 16,), jnp.float32),
        pltpu.VMEM((_CAP,), jnp.float32),
        pltpu.VMEM((_WCAP,), jnp.int32),
        pltpu.VMEM((16,), jnp.float32),
    ],
)
def _sc_thresholds(logits_hbm, targets_hbm, out_hbm, row_v, max_v, cand_v, work_v, res_v):
    _sc_body(logits_hbm, targets_hbm, out_hbm, row_v, max_v, cand_v, work_v, res_v)


_BR = 8


def _tc_body(lg_ref, tg_ref, thl_ref, tht_ref, s_ref, c_ref):
    i = pl.program_id(0)
    x = lg_ref[...]
    t = tg_ref[...]
    thl = thl_ref[...]
    tht = tht_ref[...]

    bce = jnp.maximum(x, 0.0) - x * t + jnp.log1p(jnp.exp(-jnp.abs(x)))
    tmask = (t >= tht).astype(jnp.float32)
    w = jnp.where(t > 0.5, _PRESENCE_W, _ABSENCE_W) + 0.5 * tmask
    ov = tmask * (x >= thl).astype(jnp.float32)
    s = jnp.sum(bce * w)
    cv = jnp.sum(ov)

    @pl.when(i == 0)
    def _():
        s_ref[...] = jnp.zeros((1, 1), jnp.float32)
        c_ref[...] = jnp.zeros((1, 1), jnp.float32)

    s_ref[...] += jnp.full((1, 1), 0.0) + s
    c_ref[...] += jnp.full((1, 1), 0.0) + cv


def _tc_combine(logits, targets, thl, tht):
    return pl.pallas_call(
        _tc_body,
        grid=(_B // _BR,),
        in_specs=[
            pl.BlockSpec((_BR, _N), lambda i: (i, 0)),
            pl.BlockSpec((_BR, _N), lambda i: (i, 0)),
            pl.BlockSpec((_BR, 1), lambda i: (i, 0)),
            pl.BlockSpec((_BR, 1), lambda i: (i, 0)),
        ],
        out_specs=[
            pl.BlockSpec((1, 1), lambda i: (0, 0)),
            pl.BlockSpec((1, 1), lambda i: (0, 0)),
        ],
        out_shape=[
            jax.ShapeDtypeStruct((1, 1), jnp.float32),
            jax.ShapeDtypeStruct((1, 1), jnp.float32),
        ],
    )(logits, targets, thl, tht)


def kernel(logits, targets):
    thr = _sc_thresholds(logits, targets)
    flat = thr.T.reshape(-1)[:_TASKS]
    thl = flat[:_B].reshape(_B, 1)
    tht = flat[_B:].reshape(_B, 1)
    s, c = _tc_combine(logits, targets, thl, tht)
    bce_loss = s[0, 0] / (_B * _N)
    topk_loss = 1.0 - c[0, 0] / (_K * _B)
    return bce_loss + _TOPK_BONUS_W * topk_loss
